# baseline (device time: 44899 ns/iter reference)
import jax
import jax.numpy as jnp
from jax import lax
from jax.experimental import pallas as pl
from jax.experimental.pallas import tpu as pltpu

N_DEV = 4
BLOCK = 64
DH = 64
B = 2
SQ = 512
DM = 768
HALF = DM // 2
H = 8


def _fused_body(
    x_ref, wq_ref, k_ref, v_ref, wo_ref, out_ref,
    pa, pb, acc_a, acc_b, ra1, rb1, ra2, rb2,
    send_sems, recv_sems,
):
    my = lax.axis_index("i")
    p_y = my ^ 1
    p_x = 3 - my
    bf = jnp.bfloat16
    f32 = jnp.float32

    barrier = pltpu.get_barrier_semaphore()
    for nbr in [p_y, p_x]:
        pl.semaphore_signal(
            barrier, inc=1, device_id=(nbr,), device_id_type=pl.DeviceIdType.MESH
        )

    qi = lax.broadcasted_iota(jnp.int32, (SQ, SQ), 0) // BLOCK
    kj = lax.broadcasted_iota(jnp.int32, (SQ, SQ), 1) // BLOCK
    mask = (qi == kj) | (kj == 0) | ((qi + kj) % 3 == 0)

    def compute_chunk(b):
        r0 = b * SQ
        q = jnp.dot(
            x_ref[b], wq_ref[...], preferred_element_type=f32
        )
        ctx_heads = []
        for h in range(H):
            qh = q[:, h * DH:(h + 1) * DH].astype(bf)
            kh = k_ref[b, :, h, :]
            s = jnp.dot(qh, kh.T, preferred_element_type=f32) * 0.125
            s = jnp.where(mask, s, -1e9)
            s = s - s.max(axis=-1, keepdims=True)
            e = jnp.exp(s)
            w = (e / e.sum(axis=-1, keepdims=True)).astype(bf)
            ctx_heads.append(
                jnp.dot(w, v_ref[b, :, h, :], preferred_element_type=f32)
            )
        ctx = jnp.concatenate(ctx_heads, axis=1).astype(bf)
        part = jnp.dot(ctx, wo_ref[...], preferred_element_type=f32)
        pa[r0:r0 + SQ, :] = part[:, :HALF].astype(bf)
        pb[r0:r0 + SQ, :] = part[:, HALF:].astype(bf)

    def phase1(b):
        r0 = b * SQ
        ca = pltpu.make_async_remote_copy(
            src_ref=pa.at[r0:r0 + SQ, :], dst_ref=ra1.at[r0:r0 + SQ, :],
            send_sem=send_sems.at[4 * b + 0], recv_sem=recv_sems.at[4 * b + 0],
            device_id=(p_y,), device_id_type=pl.DeviceIdType.MESH,
        )
        cb = pltpu.make_async_remote_copy(
            src_ref=pb.at[r0:r0 + SQ, :], dst_ref=rb1.at[r0:r0 + SQ, :],
            send_sem=send_sems.at[4 * b + 1], recv_sem=recv_sems.at[4 * b + 1],
            device_id=(p_x,), device_id_type=pl.DeviceIdType.MESH,
        )
        ca.start()
        cb.start()
        return ca, cb

    def phase2(b):
        r0 = b * SQ
        ca = pltpu.make_async_remote_copy(
            src_ref=acc_a.at[r0:r0 + SQ, :], dst_ref=ra2.at[r0:r0 + SQ, :],
            send_sem=send_sems.at[4 * b + 2], recv_sem=recv_sems.at[4 * b + 2],
            device_id=(p_x,), device_id_type=pl.DeviceIdType.MESH,
        )
        cb = pltpu.make_async_remote_copy(
            src_ref=acc_b.at[r0:r0 + SQ, :], dst_ref=rb2.at[r0:r0 + SQ, :],
            send_sem=send_sems.at[4 * b + 3], recv_sem=recv_sems.at[4 * b + 3],
            device_id=(p_y,), device_id_type=pl.DeviceIdType.MESH,
        )
        ca.start()
        cb.start()
        return ca, cb

    def add1(b):
        r0 = b * SQ
        sl = pl.ds(r0, SQ)
        acc_a[sl, :] = pa[sl, :] + ra1[sl, :]
        acc_b[sl, :] = pb[sl, :] + rb1[sl, :]

    def emit(b):
        r0 = b * SQ
        sl = pl.ds(r0, SQ)
        out_ref[sl, :HALF] = acc_a[sl, :].astype(f32) + ra2[sl, :].astype(f32)
        out_ref[sl, HALF:] = acc_b[sl, :].astype(f32) + rb2[sl, :].astype(f32)

    compute_chunk(0)
    pl.semaphore_wait(barrier, 2)
    p1_0 = phase1(0)
    compute_chunk(1)
    p1_1 = phase1(1)
    p1_0[0].wait()
    p1_0[1].wait()
    add1(0)
    p2_0 = phase2(0)
    p1_1[0].wait()
    p1_1[1].wait()
    add1(1)
    p2_1 = phase2(1)
    p2_0[0].wait()
    p2_0[1].wait()
    emit(0)
    p2_1[0].wait()
    p2_1[1].wait()
    emit(1)


def kernel(x, Wq, K_ext, V_ext, Wo):
    bf = jnp.bfloat16
    my = lax.axis_index("i")
    K = lax.dynamic_slice_in_dim(K_ext, my * H, H, axis=2).astype(bf)
    V = lax.dynamic_slice_in_dim(V_ext, my * H, H, axis=2).astype(bf)

    m = B * SQ
    half_buf = pltpu.VMEM((m, HALF), bf)
    out = pl.pallas_call(
        _fused_body,
        out_shape=jax.ShapeDtypeStruct((m, DM), jnp.float32),
        in_specs=[pl.BlockSpec(memory_space=pltpu.VMEM)] * 5,
        out_specs=pl.BlockSpec(memory_space=pltpu.VMEM),
        scratch_shapes=[
            half_buf, half_buf,
            half_buf, half_buf,
            half_buf, half_buf, half_buf, half_buf,
            pltpu.SemaphoreType.DMA((8,)),
            pltpu.SemaphoreType.DMA((8,)),
        ],
        compiler_params=pltpu.CompilerParams(collective_id=0),
    )(x.astype(bf), Wq.astype(bf), K, V, Wo.astype(bf))
    return out.reshape(B, SQ, DM)


# device time: 42374 ns/iter; 1.0596x vs baseline; 1.0596x over previous
import jax
import jax.numpy as jnp
from jax import lax
from jax.experimental import pallas as pl
from jax.experimental.pallas import tpu as pltpu

N_DEV = 4
BLOCK = 64
DH = 64
B = 2
SQ = 512
DM = 768
HALF = DM // 2
H = 8


def _fused_body(
    x_ref, wq_ref, k_ref, v_ref, wo_ref, out_ref,
    pa, pb, acc_a, acc_b, ra1, rb1, ra2, rb2,
    send_sems, recv_sems,
):
    my = lax.axis_index("i")
    p_y = my ^ 1
    p_x = 3 - my
    bf = jnp.bfloat16
    f32 = jnp.float32

    barrier = pltpu.get_barrier_semaphore()
    for nbr in [p_y, p_x]:
        pl.semaphore_signal(
            barrier, inc=1, device_id=(nbr,), device_id_type=pl.DeviceIdType.MESH
        )

    qi = lax.broadcasted_iota(jnp.int32, (SQ, SQ), 0) // BLOCK
    kj = lax.broadcasted_iota(jnp.int32, (SQ, SQ), 1) // BLOCK
    mask = (qi == kj) | (kj == 0) | ((qi + kj) % 3 == 0)
    bias = jnp.where(mask, 0.0, -1e9).astype(f32)

    def compute_chunk(b):
        r0 = b * SQ
        q = jnp.dot(
            x_ref[b], wq_ref[...], preferred_element_type=f32
        )
        ctx_heads = []
        for h in range(H):
            qh = q[:, h * DH:(h + 1) * DH].astype(bf)
            kh = k_ref[b, :, h, :]
            s = jnp.dot(qh, kh.T, preferred_element_type=f32) * 0.125 + bias
            e = jnp.exp(s)
            w = (e * (1.0 / e.sum(axis=-1, keepdims=True))).astype(bf)
            ctx_heads.append(
                jnp.dot(w, v_ref[b, :, h, :], preferred_element_type=f32)
            )
        ctx = jnp.concatenate(ctx_heads, axis=1).astype(bf)
        part = jnp.dot(ctx, wo_ref[...], preferred_element_type=f32)
        pa[r0:r0 + SQ, :] = part[:, :HALF].astype(bf)
        pb[r0:r0 + SQ, :] = part[:, HALF:].astype(bf)

    def phase1(b):
        r0 = b * SQ
        ca = pltpu.make_async_remote_copy(
            src_ref=pa.at[r0:r0 + SQ, :], dst_ref=ra1.at[r0:r0 + SQ, :],
            send_sem=send_sems.at[4 * b + 0], recv_sem=recv_sems.at[4 * b + 0],
            device_id=(p_y,), device_id_type=pl.DeviceIdType.MESH,
        )
        cb = pltpu.make_async_remote_copy(
            src_ref=pb.at[r0:r0 + SQ, :], dst_ref=rb1.at[r0:r0 + SQ, :],
            send_sem=send_sems.at[4 * b + 1], recv_sem=recv_sems.at[4 * b + 1],
            device_id=(p_x,), device_id_type=pl.DeviceIdType.MESH,
        )
        ca.start()
        cb.start()
        return ca, cb

    def phase2(b):
        r0 = b * SQ
        ca = pltpu.make_async_remote_copy(
            src_ref=acc_a.at[r0:r0 + SQ, :], dst_ref=ra2.at[r0:r0 + SQ, :],
            send_sem=send_sems.at[4 * b + 2], recv_sem=recv_sems.at[4 * b + 2],
            device_id=(p_x,), device_id_type=pl.DeviceIdType.MESH,
        )
        cb = pltpu.make_async_remote_copy(
            src_ref=acc_b.at[r0:r0 + SQ, :], dst_ref=rb2.at[r0:r0 + SQ, :],
            send_sem=send_sems.at[4 * b + 3], recv_sem=recv_sems.at[4 * b + 3],
            device_id=(p_y,), device_id_type=pl.DeviceIdType.MESH,
        )
        ca.start()
        cb.start()
        return ca, cb

    def add1(b):
        r0 = b * SQ
        sl = pl.ds(r0, SQ)
        acc_a[sl, :] = pa[sl, :] + ra1[sl, :]
        acc_b[sl, :] = pb[sl, :] + rb1[sl, :]

    def emit(b):
        r0 = b * SQ
        sl = pl.ds(r0, SQ)
        out_ref[sl, :HALF] = acc_a[sl, :].astype(f32) + ra2[sl, :].astype(f32)
        out_ref[sl, HALF:] = acc_b[sl, :].astype(f32) + rb2[sl, :].astype(f32)

    compute_chunk(0)
    pl.semaphore_wait(barrier, 2)
    p1_0 = phase1(0)
    compute_chunk(1)
    p1_1 = phase1(1)
    p1_0[0].wait()
    p1_0[1].wait()
    add1(0)
    p2_0 = phase2(0)
    p1_1[0].wait()
    p1_1[1].wait()
    add1(1)
    p2_1 = phase2(1)
    p2_0[0].wait()
    p2_0[1].wait()
    emit(0)
    p2_1[0].wait()
    p2_1[1].wait()
    emit(1)


def kernel(x, Wq, K_ext, V_ext, Wo):
    bf = jnp.bfloat16
    my = lax.axis_index("i")
    K = lax.dynamic_slice_in_dim(K_ext, my * H, H, axis=2).astype(bf)
    V = lax.dynamic_slice_in_dim(V_ext, my * H, H, axis=2).astype(bf)

    m = B * SQ
    half_buf = pltpu.VMEM((m, HALF), bf)
    out = pl.pallas_call(
        _fused_body,
        out_shape=jax.ShapeDtypeStruct((m, DM), jnp.float32),
        in_specs=[pl.BlockSpec(memory_space=pltpu.VMEM)] * 5,
        out_specs=pl.BlockSpec(memory_space=pltpu.VMEM),
        scratch_shapes=[
            half_buf, half_buf,
            half_buf, half_buf,
            half_buf, half_buf, half_buf, half_buf,
            pltpu.SemaphoreType.DMA((8,)),
            pltpu.SemaphoreType.DMA((8,)),
        ],
        compiler_params=pltpu.CompilerParams(collective_id=0),
    )(x.astype(bf), Wq.astype(bf), K, V, Wo.astype(bf))
    return out.reshape(B, SQ, DM)
